# baseline (speedup 1.0000x reference)
import jax
import jax.numpy as jnp
from jax.experimental import pallas as pl

_TILE_ROWS = 40


def _add_body(pos_ref, x_ref, o_ref):
    x = x_ref[...]
    pos = pos_ref[...]
    r = x.shape[0]
    o_ref[...] = (x.reshape(r // _TILE_ROWS, _TILE_ROWS, x.shape[1])
                  + pos[None]).reshape(r, x.shape[1])


def kernel(patch, pos_emb):
    B, P, D = patch.shape
    x = patch.reshape(B * P, D)
    pos40 = jnp.tile(pos_emb, (_TILE_ROWS // P, 1))
    R = 2560
    grid = ((B * P) // R,)
    out = pl.pallas_call(
        _add_body,
        grid=grid,
        in_specs=[
            pl.BlockSpec((_TILE_ROWS, D), lambda i: (0, 0)),
            pl.BlockSpec((R, D), lambda i: (i, 0)),
        ],
        out_specs=pl.BlockSpec((R, D), lambda i: (i, 0)),
        out_shape=jax.ShapeDtypeStruct((B * P, D), patch.dtype),
    )(pos40, x)
    return out.reshape(B, P, D)


# speedup vs baseline: 1.6635x; 1.6635x over previous
import jax
import jax.numpy as jnp
from jax.experimental import pallas as pl


def _add_body(pos_ref, x_ref, o_ref):
    o_ref[...] = x_ref[...] + pos_ref[...][None]


def kernel(patch, pos_emb):
    B, P, D = patch.shape
    BK = 256
    out = pl.pallas_call(
        _add_body,
        grid=(B // BK,),
        in_specs=[
            pl.BlockSpec((P, D), lambda i: (0, 0)),
            pl.BlockSpec((BK, P, D), lambda i: (i, 0, 0)),
        ],
        out_specs=pl.BlockSpec((BK, P, D), lambda i: (i, 0, 0)),
        out_shape=jax.ShapeDtypeStruct((B, P, D), patch.dtype),
    )(pos_emb, patch)
    return out
